# initial kernel scaffold (unmeasured)
import jax
import jax.numpy as jnp
from jax import lax
from jax.experimental import pallas as pl
from jax.experimental.pallas import tpu as pltpu

N_DEV = 4
BLK = 64


def kernel(x, Wq, K_ext, V_ext, Wo):
    B, Sq_sh, D = x.shape
    _, Skv, Hq, Dh = K_ext.shape
    F_sh = Wq.shape[1]
    H_sh = F_sh // Dh
    F = N_DEV * F_sh

    def body(x_ref, wq_ref, k_ref, v_ref, wo_ref, out_ref,
             wq_full, wo_full, wq_comm, wo_comm,
             qsend, qrecv, osend, orecv):
        my = lax.axis_index("i")
        left = lax.rem(my + N_DEV - 1, N_DEV)
        right = lax.rem(my + 1, N_DEV)

        barrier_sem = pltpu.get_barrier_semaphore()
        for nbr in (left, right):
            pl.semaphore_signal(
                barrier_sem, inc=1,
                device_id=(nbr,), device_id_type=pl.DeviceIdType.MESH,
            )
        pl.semaphore_wait(barrier_sem, 2)

        wq_full[:, pl.ds(my * F_sh, F_sh)] = wq_ref[...]
        wo_full[pl.ds(my * F_sh, F_sh), :] = wo_ref[...]
        wq_comm[0] = wq_ref[...]
        wo_comm[0] = wo_ref[...]

        for h in range(N_DEV - 1):
            ss = h % 2
            rs = (h + 1) % 2
            rq = pltpu.make_async_remote_copy(
                src_ref=wq_comm.at[ss], dst_ref=wq_comm.at[rs],
                send_sem=qsend.at[ss], recv_sem=qrecv.at[rs],
                device_id=(right,), device_id_type=pl.DeviceIdType.MESH,
            )
            ro = pltpu.make_async_remote_copy(
                src_ref=wo_comm.at[ss], dst_ref=wo_comm.at[rs],
                send_sem=osend.at[ss], recv_sem=orecv.at[rs],
                device_id=(right,), device_id_type=pl.DeviceIdType.MESH,
            )
            rq.start()
            ro.start()
            rq.wait()
            ro.wait()
            origin = lax.rem(my + N_DEV - 1 - h, N_DEV)
            wq_full[:, pl.ds(origin * F_sh, F_sh)] = wq_comm[rs]
            wo_full[pl.ds(origin * F_sh, F_sh), :] = wo_comm[rs]

        i_idx = lax.broadcasted_iota(jnp.int32, (Sq_sh, Skv), 0)
        j_idx = lax.broadcasted_iota(jnp.int32, (Sq_sh, Skv), 1)
        qb = my * (Sq_sh // BLK) + i_idx // BLK
        kb = j_idx // BLK
        mask = (qb == kb) | (kb == 0) | (lax.rem(qb + kb, 3) == 0)
        bias = jnp.where(mask, 0.0, -1e9).astype(jnp.float32)

        x2d = x_ref[...].reshape(B * Sq_sh, D).astype(jnp.bfloat16)

        outs = [jnp.zeros((Sq_sh, D), jnp.float32) for _ in range(B)]
        for o in range(N_DEV):
            wq_o = wq_full[:, o * F_sh:(o + 1) * F_sh].astype(jnp.bfloat16)
            wo_o = wo_full[o * F_sh:(o + 1) * F_sh, :].astype(jnp.bfloat16)
            q = jnp.dot(x2d, wq_o, preferred_element_type=jnp.float32)
            q = q.reshape(B, Sq_sh, H_sh, Dh)
            k_o = k_ref[:, :, o * H_sh:(o + 1) * H_sh, :]
            v_o = v_ref[:, :, o * H_sh:(o + 1) * H_sh, :]
            for b in range(B):
                ctx_heads = []
                for hh in range(H_sh):
                    q_bh = q[b, :, hh, :].astype(jnp.bfloat16)
                    k_bh = k_o[b, :, hh, :].astype(jnp.bfloat16)
                    v_bh = v_o[b, :, hh, :].astype(jnp.bfloat16)
                    s = lax.dot_general(
                        q_bh, k_bh, (((1,), (1,)), ((), ())),
                        preferred_element_type=jnp.float32,
                    )
                    s = s * 0.125 + bias
                    m = jnp.max(s, axis=-1, keepdims=True)
                    w = jnp.exp(s - m)
                    w = w / jnp.sum(w, axis=-1, keepdims=True)
                    ctx = lax.dot_general(
                        w.astype(jnp.bfloat16), v_bh,
                        (((1,), (0,)), ((), ())),
                        preferred_element_type=jnp.float32,
                    )
                    ctx_heads.append(ctx.astype(jnp.bfloat16))
                ctx_o = jnp.concatenate(ctx_heads, axis=1)
                outs[b] = outs[b] + jnp.dot(
                    ctx_o, wo_o, preferred_element_type=jnp.float32
                )
        for b in range(B):
            out_ref[b] = outs[b]

    return pl.pallas_call(
        body,
        out_shape=jax.ShapeDtypeStruct((B, Sq_sh, D), jnp.float32),
        in_specs=[pl.BlockSpec(memory_space=pltpu.VMEM)] * 5,
        out_specs=pl.BlockSpec(memory_space=pltpu.VMEM),
        scratch_shapes=[
            pltpu.VMEM((D, F), jnp.float32),
            pltpu.VMEM((F, D), jnp.float32),
            pltpu.VMEM((2, D, F_sh), jnp.float32),
            pltpu.VMEM((2, F_sh, D), jnp.float32),
            pltpu.SemaphoreType.DMA((2,)),
            pltpu.SemaphoreType.DMA((2,)),
            pltpu.SemaphoreType.DMA((2,)),
            pltpu.SemaphoreType.DMA((2,)),
        ],
        compiler_params=pltpu.CompilerParams(collective_id=0),
    )(x, Wq, K_ext, V_ext, Wo)


# baseline (device time: 176884 ns/iter reference)
import jax
import jax.numpy as jnp
from jax import lax
from jax.experimental import pallas as pl
from jax.experimental.pallas import tpu as pltpu

N_DEV = 4
BLK = 64


def kernel(x, Wq, K_ext, V_ext, Wo):
    B, Sq_sh, D = x.shape
    _, Skv, Hq, Dh = K_ext.shape
    F_sh = Wq.shape[1]
    H_sh = F_sh // Dh
    F = N_DEV * F_sh

    def body(x_ref, wq_ref, k_ref, v_ref, wo_ref, out_ref,
             wq_full, wo_full, wq_comm, wo_comm,
             qsend, qrecv, osend, orecv):
        my = lax.axis_index("i")
        left = lax.rem(my + N_DEV - 1, N_DEV)
        right = lax.rem(my + 1, N_DEV)

        barrier_sem = pltpu.get_barrier_semaphore()
        for nbr in (left, right):
            pl.semaphore_signal(
                barrier_sem, inc=1,
                device_id=(nbr,), device_id_type=pl.DeviceIdType.MESH,
            )
        pl.semaphore_wait(barrier_sem, 2)

        wq_bf = wq_ref[...].astype(jnp.bfloat16)
        wo_bf = wo_ref[...].astype(jnp.bfloat16)
        wq_full[:, pl.ds(my * F_sh, F_sh)] = wq_bf
        wo_full[pl.ds(my * F_sh, F_sh), :] = wo_bf
        wq_comm[0] = wq_bf
        wo_comm[0] = wo_bf

        for h in range(N_DEV - 1):
            ss = h % 2
            rs = (h + 1) % 2
            rq = pltpu.make_async_remote_copy(
                src_ref=wq_comm.at[ss], dst_ref=wq_comm.at[rs],
                send_sem=qsend.at[ss], recv_sem=qrecv.at[rs],
                device_id=(right,), device_id_type=pl.DeviceIdType.MESH,
            )
            ro = pltpu.make_async_remote_copy(
                src_ref=wo_comm.at[ss], dst_ref=wo_comm.at[rs],
                send_sem=osend.at[ss], recv_sem=orecv.at[rs],
                device_id=(right,), device_id_type=pl.DeviceIdType.MESH,
            )
            rq.start()
            ro.start()
            rq.wait()
            ro.wait()
            origin = lax.rem(my + N_DEV - 1 - h, N_DEV)
            wq_full[:, pl.ds(origin * F_sh, F_sh)] = wq_comm[rs]
            wo_full[pl.ds(origin * F_sh, F_sh), :] = wo_comm[rs]

        i_idx = lax.broadcasted_iota(jnp.int32, (Sq_sh, Skv), 0)
        j_idx = lax.broadcasted_iota(jnp.int32, (Sq_sh, Skv), 1)
        qb = my * (Sq_sh // BLK) + i_idx // BLK
        kb = j_idx // BLK
        mask = (qb == kb) | (kb == 0) | (lax.rem(qb + kb, 3) == 0)
        bias = jnp.where(mask, 0.0, -1e9).astype(jnp.float32)

        xb_bf = [x_ref[b].astype(jnp.bfloat16) for b in range(B)]

        outs = [jnp.zeros((Sq_sh, D), jnp.float32) for _ in range(B)]
        for o in range(N_DEV):
            wq_o = wq_full[:, o * F_sh:(o + 1) * F_sh]
            wo_o = wo_full[o * F_sh:(o + 1) * F_sh, :]
            for b in range(B):
                q_b = jnp.dot(
                    xb_bf[b], wq_o, preferred_element_type=jnp.float32
                ).astype(jnp.bfloat16)
                ctx_heads = []
                for hh in range(H_sh):
                    q_bh = q_b[:, hh * Dh:(hh + 1) * Dh]
                    k_bh = k_ref[b, :, o * H_sh + hh, :].astype(jnp.bfloat16)
                    v_bh = v_ref[b, :, o * H_sh + hh, :].astype(jnp.bfloat16)
                    s = lax.dot_general(
                        q_bh, k_bh, (((1,), (1,)), ((), ())),
                        preferred_element_type=jnp.float32,
                    )
                    s = s * 0.125 + bias
                    m = jnp.max(s, axis=-1, keepdims=True)
                    w = jnp.exp(s - m)
                    w = w / jnp.sum(w, axis=-1, keepdims=True)
                    ctx = lax.dot_general(
                        w.astype(jnp.bfloat16), v_bh,
                        (((1,), (0,)), ((), ())),
                        preferred_element_type=jnp.float32,
                    )
                    ctx_heads.append(ctx.astype(jnp.bfloat16))
                ctx_o = jnp.concatenate(ctx_heads, axis=1)
                outs[b] = outs[b] + jnp.dot(
                    ctx_o, wo_o, preferred_element_type=jnp.float32
                )
        for b in range(B):
            out_ref[b] = outs[b]

    return pl.pallas_call(
        body,
        out_shape=jax.ShapeDtypeStruct((B, Sq_sh, D), jnp.float32),
        in_specs=[pl.BlockSpec(memory_space=pltpu.VMEM)] * 5,
        out_specs=pl.BlockSpec(memory_space=pltpu.VMEM),
        scratch_shapes=[
            pltpu.VMEM((D, F), jnp.bfloat16),
            pltpu.VMEM((F, D), jnp.bfloat16),
            pltpu.VMEM((2, D, F_sh), jnp.bfloat16),
            pltpu.VMEM((2, F_sh, D), jnp.bfloat16),
            pltpu.SemaphoreType.DMA((2,)),
            pltpu.SemaphoreType.DMA((2,)),
            pltpu.SemaphoreType.DMA((2,)),
            pltpu.SemaphoreType.DMA((2,)),
        ],
        compiler_params=pltpu.CompilerParams(
            collective_id=0, vmem_limit_bytes=62 * 1024 * 1024,
        ),
    )(x, Wq, K_ext, V_ext, Wo)


# device time: 135171 ns/iter; 1.3086x vs baseline; 1.3086x over previous
import jax
import jax.numpy as jnp
from jax import lax
from jax.experimental import pallas as pl
from jax.experimental.pallas import tpu as pltpu

N_DEV = 4
BLK = 64


def kernel(x, Wq, K_ext, V_ext, Wo):
    B, Sq_sh, D = x.shape
    _, Skv, Hq, Dh = K_ext.shape
    F_sh = Wq.shape[1]
    H_sh = F_sh // Dh
    F = N_DEV * F_sh

    def body(x_ref, wq_ref, k_ref, v_ref, wo_ref, out_ref,
             wq_comm, wo_comm, qsend, qrecv, osend, orecv):
        my = lax.axis_index("i")

        barrier_sem = pltpu.get_barrier_semaphore()
        for j in range(1, N_DEV):
            pl.semaphore_signal(
                barrier_sem, inc=1,
                device_id=(lax.rem(my + j, N_DEV),),
                device_id_type=pl.DeviceIdType.MESH,
            )
        pl.semaphore_wait(barrier_sem, N_DEV - 1)

        wq_comm[pl.ds(my, 1)] = wq_ref[...].astype(jnp.bfloat16)[None]
        wo_comm[pl.ds(my, 1)] = wo_ref[...].astype(jnp.bfloat16)[None]

        for j in range(1, N_DEV):
            tgt = lax.rem(my + j, N_DEV)
            for comm, ssem, rsem in (
                (wq_comm, qsend, qrecv), (wo_comm, osend, orecv)
            ):
                pltpu.make_async_remote_copy(
                    src_ref=comm.at[my], dst_ref=comm.at[my],
                    send_sem=ssem.at[j - 1], recv_sem=rsem.at[my],
                    device_id=(tgt,), device_id_type=pl.DeviceIdType.MESH,
                ).start()

        i_idx = lax.broadcasted_iota(jnp.int32, (Sq_sh, Skv), 0)
        j_idx = lax.broadcasted_iota(jnp.int32, (Sq_sh, Skv), 1)
        qb = my * (Sq_sh // BLK) + i_idx // BLK
        kb = j_idx // BLK
        mask = (qb == kb) | (kb == 0) | (lax.rem(qb + kb, 3) == 0)
        bias = jnp.where(mask, 0.0, -1e9).astype(jnp.float32)

        xb_bf = [x_ref[b].astype(jnp.bfloat16) for b in range(B)]
        ones_col = jnp.ones((Skv, Dh), jnp.bfloat16)

        outs = [jnp.zeros((Sq_sh, D), jnp.float32) for _ in range(B)]
        for s in (0, 1, 3, 2):
            oo = lax.rem(my + s, N_DEV)
            if s != 0:
                for comm, ssem, rsem in (
                    (wq_comm, qsend, qrecv), (wo_comm, osend, orecv)
                ):
                    pltpu.make_async_remote_copy(
                        src_ref=comm.at[oo], dst_ref=comm.at[oo],
                        send_sem=ssem.at[0], recv_sem=rsem.at[oo],
                        device_id=(my,), device_id_type=pl.DeviceIdType.MESH,
                    ).wait_recv()
            wq_o = wq_comm[oo]
            wo_o = wo_comm[oo]
            for b in range(B):
                q_b = jnp.dot(
                    xb_bf[b], wq_o, preferred_element_type=jnp.float32
                ).astype(jnp.bfloat16)
                ctx_heads = []
                for hh in range(H_sh):
                    q_bh = q_b[:, hh * Dh:(hh + 1) * Dh]
                    h_ix = oo * H_sh + hh
                    k_bh = k_ref[b, :, h_ix, :].astype(jnp.bfloat16)
                    v_bh = v_ref[b, :, h_ix, :].astype(jnp.bfloat16)
                    s_qk = lax.dot_general(
                        q_bh, k_bh, (((1,), (1,)), ((), ())),
                        preferred_element_type=jnp.float32,
                    )
                    w = jnp.exp(s_qk * 0.125 + bias).astype(jnp.bfloat16)
                    v_cat = jnp.concatenate([v_bh, ones_col], axis=1)
                    ctx2 = lax.dot_general(
                        w, v_cat, (((1,), (0,)), ((), ())),
                        preferred_element_type=jnp.float32,
                    )
                    ctx = ctx2[:, :Dh] / ctx2[:, Dh:Dh + 1]
                    ctx_heads.append(ctx.astype(jnp.bfloat16))
                ctx_o = jnp.concatenate(ctx_heads, axis=1)
                outs[b] = outs[b] + jnp.dot(
                    ctx_o, wo_o, preferred_element_type=jnp.float32
                )
        for b in range(B):
            out_ref[b] = outs[b]

        for j in range(1, N_DEV):
            tgt = lax.rem(my + j, N_DEV)
            for comm, ssem, rsem in (
                (wq_comm, qsend, qrecv), (wo_comm, osend, orecv)
            ):
                pltpu.make_async_remote_copy(
                    src_ref=comm.at[my], dst_ref=comm.at[my],
                    send_sem=ssem.at[j - 1], recv_sem=rsem.at[my],
                    device_id=(tgt,), device_id_type=pl.DeviceIdType.MESH,
                ).wait_send()

    return pl.pallas_call(
        body,
        out_shape=jax.ShapeDtypeStruct((B, Sq_sh, D), jnp.float32),
        in_specs=[pl.BlockSpec(memory_space=pltpu.VMEM)] * 5,
        out_specs=pl.BlockSpec(memory_space=pltpu.VMEM),
        scratch_shapes=[
            pltpu.VMEM((N_DEV, D, F_sh), jnp.bfloat16),
            pltpu.VMEM((N_DEV, F_sh, D), jnp.bfloat16),
            pltpu.SemaphoreType.DMA((N_DEV - 1,)),
            pltpu.SemaphoreType.DMA((N_DEV,)),
            pltpu.SemaphoreType.DMA((N_DEV - 1,)),
            pltpu.SemaphoreType.DMA((N_DEV,)),
        ],
        compiler_params=pltpu.CompilerParams(
            collective_id=0, vmem_limit_bytes=62 * 1024 * 1024,
        ),
    )(x, Wq, K_ext, V_ext, Wo)


# device time: 116586 ns/iter; 1.5172x vs baseline; 1.1594x over previous
import jax
import jax.numpy as jnp
from jax import lax
from jax.experimental import pallas as pl
from jax.experimental.pallas import tpu as pltpu

N_DEV = 4
BLK = 64


def kernel(x, Wq, K_ext, V_ext, Wo):
    B, Sq_sh, D = x.shape
    _, Skv, Hq, Dh = K_ext.shape
    F_sh = Wq.shape[1]
    H_sh = F_sh // Dh
    F = N_DEV * F_sh

    def body(x_ref, wq_ref, k_ref, v_ref, wo_ref, out_ref,
             wq_comm, wo_comm, k_hm, v_hm, qsend, qrecv, osend, orecv):
        my = lax.axis_index("i")

        barrier_sem = pltpu.get_barrier_semaphore()
        for j in range(1, N_DEV):
            pl.semaphore_signal(
                barrier_sem, inc=1,
                device_id=(lax.rem(my + j, N_DEV),),
                device_id_type=pl.DeviceIdType.MESH,
            )
        pl.semaphore_wait(barrier_sem, N_DEV - 1)

        wq_comm[pl.ds(my, 1)] = wq_ref[...].astype(jnp.bfloat16)[None]
        wo_comm[pl.ds(my, 1)] = wo_ref[...].astype(jnp.bfloat16)[None]

        for j in range(1, N_DEV):
            tgt = lax.rem(my + j, N_DEV)
            for comm, ssem, rsem in (
                (wq_comm, qsend, qrecv), (wo_comm, osend, orecv)
            ):
                pltpu.make_async_remote_copy(
                    src_ref=comm.at[my], dst_ref=comm.at[my],
                    send_sem=ssem.at[j - 1], recv_sem=rsem.at[my],
                    device_id=(tgt,), device_id_type=pl.DeviceIdType.MESH,
                ).start()

        for b in range(B):
            for c in range(N_DEV):
                k_hm[b, :, c * F_sh:(c + 1) * F_sh] = (
                    k_ref[b, :, c * H_sh:(c + 1) * H_sh, :]
                    .reshape(Skv, F_sh).astype(jnp.bfloat16)
                )
                v_hm[b, :, c * F_sh:(c + 1) * F_sh] = (
                    v_ref[b, :, c * H_sh:(c + 1) * H_sh, :]
                    .reshape(Skv, F_sh).astype(jnp.bfloat16)
                )

        i_idx = lax.broadcasted_iota(jnp.int32, (Sq_sh, Skv), 0)
        j_idx = lax.broadcasted_iota(jnp.int32, (Sq_sh, Skv), 1)
        qb = my * (Sq_sh // BLK) + i_idx // BLK
        kb = j_idx // BLK
        mask = (qb == kb) | (kb == 0) | (lax.rem(qb + kb, 3) == 0)
        bias = jnp.where(mask, 0.0, -1e9).astype(jnp.float32)

        xb_bf = [x_ref[b].astype(jnp.bfloat16) for b in range(B)]
        ones_mat = jnp.ones((Skv, 8), jnp.bfloat16)

        for b in range(B):
            out_ref[b] = jnp.zeros((Sq_sh, D), jnp.float32)
        for s in (0, 1, 3, 2):
            oo = lax.rem(my + s, N_DEV)
            if s != 0:
                for comm, ssem, rsem in (
                    (wq_comm, qsend, qrecv), (wo_comm, osend, orecv)
                ):
                    pltpu.make_async_remote_copy(
                        src_ref=comm.at[oo], dst_ref=comm.at[oo],
                        send_sem=ssem.at[0], recv_sem=rsem.at[oo],
                        device_id=(my,), device_id_type=pl.DeviceIdType.MESH,
                    ).wait_recv()
            wq_o = wq_comm[oo]
            wo_o = wo_comm[oo]
            for b in range(B):
                q_b = jnp.dot(
                    xb_bf[b], wq_o, preferred_element_type=jnp.float32
                ).astype(jnp.bfloat16)
                k_blk = k_hm[b, :, pl.ds(oo * F_sh, F_sh)]
                v_blk = v_hm[b, :, pl.ds(oo * F_sh, F_sh)]
                ctx_heads = []
                for hh in range(H_sh):
                    q_bh = q_b[:, hh * Dh:(hh + 1) * Dh]
                    k_bh = k_blk[:, hh * Dh:(hh + 1) * Dh]
                    v_bh = v_blk[:, hh * Dh:(hh + 1) * Dh]
                    s_qk = lax.dot_general(
                        q_bh, k_bh, (((1,), (1,)), ((), ())),
                        preferred_element_type=jnp.float32,
                    )
                    w = jnp.exp(s_qk * 0.125 + bias).astype(jnp.bfloat16)
                    ctx = lax.dot_general(
                        w, v_bh, (((1,), (0,)), ((), ())),
                        preferred_element_type=jnp.float32,
                    )
                    den = lax.dot_general(
                        w, ones_mat, (((1,), (0,)), ((), ())),
                        preferred_element_type=jnp.float32,
                    )
                    ctx = ctx / den[:, 0:1]
                    ctx_heads.append(ctx.astype(jnp.bfloat16))
                ctx_o = jnp.concatenate(ctx_heads, axis=1)
                out_ref[b] = out_ref[b] + jnp.dot(
                    ctx_o, wo_o, preferred_element_type=jnp.float32
                )

        for j in range(1, N_DEV):
            tgt = lax.rem(my + j, N_DEV)
            for comm, ssem, rsem in (
                (wq_comm, qsend, qrecv), (wo_comm, osend, orecv)
            ):
                pltpu.make_async_remote_copy(
                    src_ref=comm.at[my], dst_ref=comm.at[my],
                    send_sem=ssem.at[j - 1], recv_sem=rsem.at[my],
                    device_id=(tgt,), device_id_type=pl.DeviceIdType.MESH,
                ).wait_send()

    return pl.pallas_call(
        body,
        out_shape=jax.ShapeDtypeStruct((B, Sq_sh, D), jnp.float32),
        in_specs=[pl.BlockSpec(memory_space=pltpu.VMEM)] * 5,
        out_specs=pl.BlockSpec(memory_space=pltpu.VMEM),
        scratch_shapes=[
            pltpu.VMEM((N_DEV, D, F_sh), jnp.bfloat16),
            pltpu.VMEM((N_DEV, F_sh, D), jnp.bfloat16),
            pltpu.VMEM((B, Skv, F), jnp.bfloat16),
            pltpu.VMEM((B, Skv, F), jnp.bfloat16),
            pltpu.SemaphoreType.DMA((N_DEV - 1,)),
            pltpu.SemaphoreType.DMA((N_DEV,)),
            pltpu.SemaphoreType.DMA((N_DEV - 1,)),
            pltpu.SemaphoreType.DMA((N_DEV,)),
        ],
        compiler_params=pltpu.CompilerParams(
            collective_id=0, vmem_limit_bytes=62 * 1024 * 1024,
        ),
    )(x, Wq, K_ext, V_ext, Wo)
